# Initial kernel scaffold; baseline (speedup 1.0000x reference)
import jax
import jax.numpy as jnp
from jax.experimental import pallas as pl


def kernel(x, champ_table, item_table, trait_table):
    raise NotImplementedError("write your pallas kernel here")


# speedup vs baseline: 12.3412x; 12.3412x over previous
import functools

import jax
import jax.numpy as jnp
from jax import lax
from jax.experimental import pallas as pl
from jax.experimental.pallas import tpu as pltpu
from jax.experimental.pallas import tpu_sc as plsc

B, L, C = 16384, 50, 23
N = B * L
OUT_D = 128
CHAMP_D, ITEM_D, TRAIT_D = 30, 10, 8
CH_BASE = 0
IT_BASE = 60 * CHAMP_D
TR_BASE = IT_BASE + 60 * ITEM_D
TBL_N = TR_BASE + 27 * TRAIT_D

NC, NS = 2, 16
NW = NC * NS
TOK_PER_W = N // NW
T = 256
CHUNKS = TOK_PER_W // T


def _body(x_hbm, tbl_hbm, out_hbm, xv, outv, tblv):
    c = lax.axis_index("c")
    s = lax.axis_index("s")
    wid = s * NC + c
    pltpu.sync_copy(tbl_hbm, tblv)
    iota = lax.broadcasted_iota(jnp.int32, (16,), 0)

    def g_body(g, carry):
        row = iota + g * 16
        xoff = row * C
        ooff = row * OUT_D
        ids = []
        for slot in range(11):
            f = plsc.load_gather(xv, [xoff + slot])
            ids.append(f.astype(jnp.int32))
        b0 = ids[0] * CHAMP_D + CH_BASE
        for j in range(CHAMP_D):
            v = plsc.load_gather(tblv, [b0 + j])
            plsc.store_scatter(outv, [ooff + j], v)
        for k in range(3):
            bk = ids[1 + k] * ITEM_D + IT_BASE
            for j in range(ITEM_D):
                v = plsc.load_gather(tblv, [bk + j])
                plsc.store_scatter(outv, [ooff + (CHAMP_D + k * ITEM_D + j)], v)
        for k in range(7):
            bk = ids[4 + k] * TRAIT_D + TR_BASE
            for j in range(TRAIT_D):
                v = plsc.load_gather(tblv, [bk + j])
                plsc.store_scatter(
                    outv, [ooff + (CHAMP_D + 30 + k * TRAIT_D + j)], v)
        for j in range(12):
            v = plsc.load_gather(xv, [xoff + 11 + j])
            plsc.store_scatter(outv, [ooff + (116 + j)], v)
        return carry

    def chunk_body(ci, carry):
        base = (wid * CHUNKS + ci) * T
        pltpu.sync_copy(x_hbm.at[pl.ds(base * C, T * C)], xv)
        lax.fori_loop(0, T // 16, g_body, None)
        pltpu.sync_copy(outv, out_hbm.at[pl.ds(base * OUT_D, T * OUT_D)])
        return carry

    lax.fori_loop(0, CHUNKS, chunk_body, None)


@jax.jit
def kernel(x, champ_table, item_table, trait_table):
    x_flat = x.reshape(N * C)
    tbl = jnp.concatenate([
        champ_table.reshape(-1),
        item_table.reshape(-1),
        trait_table.reshape(-1),
    ])
    mesh = plsc.VectorSubcoreMesh(core_axis_name="c", subcore_axis_name="s")
    f = pl.kernel(
        _body,
        out_type=jax.ShapeDtypeStruct((N * OUT_D,), jnp.float32),
        mesh=mesh,
        compiler_params=pltpu.CompilerParams(needs_layout_passes=False),
        scratch_types=[
            pltpu.VMEM((T * C,), jnp.float32),
            pltpu.VMEM((T * OUT_D,), jnp.float32),
            pltpu.VMEM((TBL_N,), jnp.float32),
        ],
    )
    out_flat = f(x_flat, tbl)
    return out_flat.reshape(B, L, OUT_D)


# speedup vs baseline: 17.0346x; 1.3803x over previous
import functools

import jax
import jax.numpy as jnp
from jax import lax
from jax.experimental import pallas as pl
from jax.experimental.pallas import tpu as pltpu
from jax.experimental.pallas import tpu_sc as plsc

B, L, C = 16384, 50, 23
N = B * L
OUT_D = 128
CHAMP_D, ITEM_D, TRAIT_D = 30, 10, 8
CH_BASE = 0
IT_BASE = 60 * CHAMP_D
TR_BASE = IT_BASE + 60 * ITEM_D
TBL_N = TR_BASE + 27 * TRAIT_D

NC, NS = 2, 16
NW = NC * NS
TOK_PER_W = N // NW
T = 256
CHUNKS = TOK_PER_W // T


def _assemble_chunk(xv_b, outv_b, tblv):
    iota = lax.broadcasted_iota(jnp.int32, (16,), 0)

    @plsc.parallel_loop(0, T // 16)
    def _(g):
        row = iota + g * 16
        xoff = row * C
        ooff = row * OUT_D
        ids = []
        for slot in range(11):
            f = plsc.load_gather(xv_b, [xoff + slot])
            ids.append(f.astype(jnp.int32))
        b0 = ids[0] * CHAMP_D + CH_BASE
        for j in range(CHAMP_D):
            v = plsc.load_gather(tblv, [b0 + j])
            plsc.store_scatter(outv_b, [ooff + j], v)
        for k in range(3):
            bk = ids[1 + k] * ITEM_D + IT_BASE
            for j in range(ITEM_D):
                v = plsc.load_gather(tblv, [bk + j])
                plsc.store_scatter(outv_b, [ooff + (CHAMP_D + k * ITEM_D + j)], v)
        for k in range(7):
            bk = ids[4 + k] * TRAIT_D + TR_BASE
            for j in range(TRAIT_D):
                v = plsc.load_gather(tblv, [bk + j])
                plsc.store_scatter(
                    outv_b, [ooff + (CHAMP_D + 30 + k * TRAIT_D + j)], v)
        for j in range(12):
            v = plsc.load_gather(xv_b, [xoff + 11 + j])
            plsc.store_scatter(outv_b, [ooff + (116 + j)], v)


def _body(x_hbm, tbl_hbm, out_hbm, xv0, xv1, ov0, ov1, tblv, xs0, xs1, os0, os1):
    c = lax.axis_index("c")
    s = lax.axis_index("s")
    wid = s * NC + c
    pltpu.sync_copy(tbl_hbm, tblv)
    xbufs = (xv0, xv1)
    obufs = (ov0, ov1)
    xsems = (xs0, xs1)
    osems = (os0, os1)

    def x_slice(ci):
        base = (wid * CHUNKS + ci) * T
        return x_hbm.at[pl.ds(base * C, T * C)]

    def o_slice(ci):
        base = (wid * CHUNKS + ci) * T
        return out_hbm.at[pl.ds(base * OUT_D, T * OUT_D)]

    pltpu.async_copy(x_slice(0), xv0, xs0)
    pltpu.async_copy(x_slice(1), xv1, xs1)

    @pl.loop(0, CHUNKS, step=2)
    def _(ci0):
        for bi in range(2):
            ci = ci0 + bi
            xv_b = xbufs[bi]
            outv_b = obufs[bi]
            pltpu.make_async_copy(x_slice(ci), xv_b, xsems[bi]).wait()

            @pl.when(ci >= 2)
            def _():
                pltpu.make_async_copy(outv_b, o_slice(ci - 2), osems[bi]).wait()

            _assemble_chunk(xv_b, outv_b, tblv)
            pltpu.async_copy(outv_b, o_slice(ci), osems[bi])

            @pl.when(ci + 2 < CHUNKS)
            def _():
                pltpu.async_copy(x_slice(ci + 2), xv_b, xsems[bi])

    pltpu.make_async_copy(ov0, o_slice(CHUNKS - 2), os0).wait()
    pltpu.make_async_copy(ov1, o_slice(CHUNKS - 1), os1).wait()


@jax.jit
def kernel(x, champ_table, item_table, trait_table):
    x_flat = x.reshape(N * C)
    tbl = jnp.concatenate([
        champ_table.reshape(-1),
        item_table.reshape(-1),
        trait_table.reshape(-1),
    ])
    mesh = plsc.VectorSubcoreMesh(core_axis_name="c", subcore_axis_name="s")
    f = pl.kernel(
        _body,
        out_type=jax.ShapeDtypeStruct((N * OUT_D,), jnp.float32),
        mesh=mesh,
        compiler_params=pltpu.CompilerParams(needs_layout_passes=False),
        scratch_types=[
            pltpu.VMEM((T * C,), jnp.float32),
            pltpu.VMEM((T * C,), jnp.float32),
            pltpu.VMEM((T * OUT_D,), jnp.float32),
            pltpu.VMEM((T * OUT_D,), jnp.float32),
            pltpu.VMEM((TBL_N,), jnp.float32),
            pltpu.SemaphoreType.DMA,
            pltpu.SemaphoreType.DMA,
            pltpu.SemaphoreType.DMA,
            pltpu.SemaphoreType.DMA,
        ],
    )
    out_flat = f(x_flat, tbl)
    return out_flat.reshape(B, L, OUT_D)


# speedup vs baseline: 31.6946x; 1.8606x over previous
import functools

import jax
import jax.numpy as jnp
from jax import lax
from jax.experimental import pallas as pl
from jax.experimental.pallas import tpu as pltpu
from jax.experimental.pallas import tpu_sc as plsc

B, L, C = 16384, 50, 23
N = B * L
OUT_D = 128
CHAMP_D, ITEM_D, TRAIT_D = 30, 10, 8
CH_BASE = 0
IT_BASE = 60 * CHAMP_D
TR_BASE = IT_BASE + 60 * ITEM_D
XBASE = TR_BASE + 27 * TRAIT_D
TBL_N = XBASE

NC, NS = 2, 16
NW = NC * NS
TOK_PER_W = N // NW
T = 256
CHUNKS = TOK_PER_W // T
NG = OUT_D // 16

def _make_col_consts():
    iota = lax.broadcasted_iota(jnp.int32, (16,), 0)
    slotmaps, strides, offs = [], [], []
    for g in range(NG):
        col = iota + 16 * g
        is_ch = col < 30
        is_it = jnp.logical_and(col >= 30, col < 60)
        is_tr = jnp.logical_and(col >= 60, col < 116)
        is_xs = col >= 116
        it_k = (col - 30) // ITEM_D
        it_j = (col - 30) - it_k * ITEM_D
        tr_k = (col - 60) // TRAIT_D
        tr_j = (col - 60) - tr_k * TRAIT_D
        slot = jnp.where(is_it, 1 + it_k, jnp.where(is_tr, 4 + tr_k, 0))
        stride = jnp.where(
            is_ch, CHAMP_D,
            jnp.where(is_it, ITEM_D, jnp.where(is_tr, TRAIT_D, 0)))
        off = jnp.where(
            is_ch, CH_BASE + col,
            jnp.where(is_it, IT_BASE + it_j,
                      jnp.where(is_tr, TR_BASE + tr_j,
                                XBASE + 11 + (col - 116))))
        slotmaps.append(slot)
        strides.append(stride)
        offs.append(off)
    xmask7 = ((iota + 16 * (NG - 1)) >= 116).astype(jnp.int32)
    return slotmaps, strides, offs, xmask7


def _assemble_chunk(buf_b, outv_b, consts):
    slotmaps, strides, offs, xmask7 = consts
    iota = lax.broadcasted_iota(jnp.int32, (16,), 0)

    @plsc.parallel_loop(0, T)
    def _(t):
        xrow = XBASE + t * C
        ids_f = plsc.load_gather(buf_b, [xrow + iota])
        ids = ids_f.astype(jnp.int32)
        obase = t * OUT_D
        for g in range(NG):
            ids_p = jnp.take_along_axis(
                ids, slotmaps[g], axis=0, mode="promise_in_bounds")
            idx = ids_p * strides[g] + offs[g]
            if g == NG - 1:
                idx = idx + xmask7 * (t * C)
            v = plsc.load_gather(buf_b, [idx])
            outv_b[pl.ds(obase + 16 * g, 16)] = v


def _body(x_hbm, tbl_hbm, out_hbm, bv0, bv1, ov0, ov1, xs0, xs1, os0, os1):
    c = lax.axis_index("c")
    s = lax.axis_index("s")
    wid = s * NC + c
    consts = _make_col_consts()
    pltpu.sync_copy(tbl_hbm, bv0.at[pl.ds(0, TBL_N)])
    pltpu.sync_copy(tbl_hbm, bv1.at[pl.ds(0, TBL_N)])
    bbufs = (bv0, bv1)
    obufs = (ov0, ov1)
    xsems = (xs0, xs1)
    osems = (os0, os1)

    def x_slice(ci):
        base = (wid * CHUNKS + ci) * T
        return x_hbm.at[pl.ds(base * C, T * C)]

    def o_slice(ci):
        base = (wid * CHUNKS + ci) * T
        return out_hbm.at[pl.ds(base * OUT_D, T * OUT_D)]

    pltpu.async_copy(x_slice(0), bv0.at[pl.ds(XBASE, T * C)], xs0)
    pltpu.async_copy(x_slice(1), bv1.at[pl.ds(XBASE, T * C)], xs1)

    @pl.loop(0, CHUNKS, step=2)
    def _(ci0):
        for bi in range(2):
            ci = ci0 + bi
            buf_b = bbufs[bi]
            outv_b = obufs[bi]
            pltpu.make_async_copy(
                x_slice(ci), buf_b.at[pl.ds(XBASE, T * C)], xsems[bi]).wait()

            @pl.when(ci >= 2)
            def _():
                pltpu.make_async_copy(outv_b, o_slice(ci - 2), osems[bi]).wait()

            _assemble_chunk(buf_b, outv_b, consts)
            pltpu.async_copy(outv_b, o_slice(ci), osems[bi])

            @pl.when(ci + 2 < CHUNKS)
            def _():
                pltpu.async_copy(
                    x_slice(ci + 2), buf_b.at[pl.ds(XBASE, T * C)], xsems[bi])

    pltpu.make_async_copy(ov0, o_slice(CHUNKS - 2), os0).wait()
    pltpu.make_async_copy(ov1, o_slice(CHUNKS - 1), os1).wait()


@jax.jit
def kernel(x, champ_table, item_table, trait_table):
    x_flat = x.reshape(N * C)
    tbl = jnp.concatenate([
        champ_table.reshape(-1),
        item_table.reshape(-1),
        trait_table.reshape(-1),
    ])
    mesh = plsc.VectorSubcoreMesh(core_axis_name="c", subcore_axis_name="s")
    f = pl.kernel(
        _body,
        out_type=jax.ShapeDtypeStruct((N * OUT_D,), jnp.float32),
        mesh=mesh,
        compiler_params=pltpu.CompilerParams(needs_layout_passes=False),
        scratch_types=[
            pltpu.VMEM((XBASE + T * C,), jnp.float32),
            pltpu.VMEM((XBASE + T * C,), jnp.float32),
            pltpu.VMEM((T * OUT_D,), jnp.float32),
            pltpu.VMEM((T * OUT_D,), jnp.float32),
            pltpu.SemaphoreType.DMA,
            pltpu.SemaphoreType.DMA,
            pltpu.SemaphoreType.DMA,
            pltpu.SemaphoreType.DMA,
        ],
    )
    out_flat = f(x_flat, tbl)
    return out_flat.reshape(B, L, OUT_D)
